# baseline (speedup 1.0000x reference)
import jax
import jax.numpy as jnp
from jax import lax
from jax.experimental import pallas as pl
from jax.experimental.pallas import tpu as pltpu, tpu_sc as plsc

_B, _T, _S, _O, _F = 16, 4, 2, 14, 4096
_NSC = 3276
_NSCP = 3328
_NTILE = _NSCP // 128
_INW = _S * _F
_NBT = _B * _T
_NW = 32
_BT_PER_W = _NBT // _NW
_LANES = 16
_NJ = _NSCP // _LANES
_CHUNKS = ((0, 4), (4, 4), (8, 4), (12, 2))
_ROWS_TR = (8, _O - 8)


def _gather_body(in_hbm, tin_hbm, out_hbm, tin_v, in_v0, in_v1, out_s0,
                 out_s1, isem0, isem1, osem):
    c = lax.axis_index("c")
    s = lax.axis_index("s")
    wid = s * 2 + c
    bt0 = wid * _BT_PER_W
    pltpu.sync_copy(tin_hbm, tin_v)

    in_bufs = (in_v0, in_v1)
    out_bufs = (out_s0, out_s1)
    in_sems = (isem0, isem1)

    def start_in(ci):
        g, (o0, no) = divmod(ci, 4)[0], _CHUNKS[ci % 4]
        base = ((bt0 + g) * _O + o0) * _INW
        return pltpu.async_copy(
            in_hbm.at[pl.ds(base, no * _INW)],
            in_bufs[ci % 2].at[pl.ds(0, no * _INW)],
            in_sems[ci % 2],
        )

    def start_out(g, tr):
        bt = bt0 + g
        b, t = bt // _T, bt % _T
        rows = _ROWS_TR[tr]
        copies = []
        for s2 in range(_S):
            copies.append(
                pltpu.async_copy(
                    out_bufs[s2].at[pl.ds(0, rows)],
                    out_hbm.at[b, t, s2, pl.ds(tr * 8, rows)],
                    osem,
                )
            )
        return copies

    def gather_chunk(in_v, o0, no):
        @plsc.parallel_loop(0, _NJ, unroll=8)
        def j_body(j):
            tin = tin_v[pl.ds(j * _LANES, _LANES)]
            c0 = j * _LANES
            for u in range(no):
                r8 = (o0 + u) % 8
                for s2 in range(_S):
                    out_bufs[s2][r8, pl.ds(c0, _LANES)] = (
                        plsc.load_gather(
                            in_v, [tin + (u * _INW + s2 * 128)]
                        )
                    )

    out_copies = None
    in_copies = [None] * (4 * _BT_PER_W)
    in_copies[0] = start_in(0)
    for g in range(_BT_PER_W):
        for tr in range(2):
            for h in range(2):
                ci = g * 4 + tr * 2 + h
                if ci + 1 < 4 * _BT_PER_W:
                    in_copies[ci + 1] = start_in(ci + 1)
                in_copies[ci].wait()
                if h == 0 and out_copies is not None:
                    for cp in out_copies:
                        cp.wait()
                o0, no = _CHUNKS[tr * 2 + h]
                gather_chunk(in_bufs[ci % 2], o0, no)
            out_copies = start_out(g, tr)
    for cp in out_copies:
        cp.wait()


def kernel(inputs, sc_ind):
    x = inputs.transpose(0, 1, 3, 2, 4).reshape(_B, _T, _O, _S, _F // 128,
                                                128)
    x = x.transpose(0, 1, 2, 4, 3, 5).reshape(_NBT * _O * _INW)
    idx32 = sc_ind.astype(jnp.int32)
    idx = jnp.concatenate(
        [idx32, jnp.full((_NSCP - _NSC,), idx32[-1], jnp.int32)]
    )
    tin = (idx >> 7) * 256 + (idx & 127)
    mesh = plsc.VectorSubcoreMesh(core_axis_name="c", subcore_axis_name="s")
    out = pl.kernel(
        _gather_body,
        mesh=mesh,
        compiler_params=pltpu.CompilerParams(
            needs_layout_passes=False, use_tc_tiling_on_sc=True
        ),
        out_type=jax.ShapeDtypeStruct((_B, _T, _S, _O, _NSCP), jnp.float32),
        scratch_types=[
            pltpu.VMEM((_NJ * _LANES,), jnp.int32),
            pltpu.VMEM((4 * _INW,), jnp.float32),
            pltpu.VMEM((4 * _INW,), jnp.float32),
            pltpu.VMEM((8, _NSCP), jnp.float32),
            pltpu.VMEM((8, _NSCP), jnp.float32),
            pltpu.SemaphoreType.DMA,
            pltpu.SemaphoreType.DMA,
            pltpu.SemaphoreType.DMA,
        ],
    )(x, tin)
    return out[..., :_NSC]


# speedup vs baseline: 1.0064x; 1.0064x over previous
import jax
import jax.numpy as jnp
from jax import lax
from jax.experimental import pallas as pl
from jax.experimental.pallas import tpu as pltpu, tpu_sc as plsc

_B, _T, _S, _O, _F = 16, 4, 2, 14, 4096
_NSC = 3276
_NSCP = 3328
_NTILE = _NSCP // 128
_INW = _S * _F
_NBT = _B * _T
_NW = 32
_BT_PER_W = _NBT // _NW
_LANES = 16
_NJ = _NSCP // _LANES
_CHUNKS = ((0, 4), (4, 4), (8, 4), (12, 2))
_ROWS_TR = (8, _O - 8)


def _gather_body(in_hbm, tin_hbm, out_hbm, tin_v, in_v0, in_v1, out_s0,
                 out_s1, isem0, isem1, osem):
    c = lax.axis_index("c")
    s = lax.axis_index("s")
    wid = s * 2 + c
    bt0 = wid * _BT_PER_W
    pltpu.sync_copy(tin_hbm, tin_v)

    in_bufs = (in_v0, in_v1)
    out_bufs = (out_s0, out_s1)
    in_sems = (isem0, isem1)

    def start_in(ci):
        g, (o0, no) = divmod(ci, 4)[0], _CHUNKS[ci % 4]
        base = ((bt0 + g) * _O + o0) * _INW
        return pltpu.async_copy(
            in_hbm.at[pl.ds(base, no * _INW)],
            in_bufs[ci % 2].at[pl.ds(0, no * _INW)],
            in_sems[ci % 2],
        )

    def start_out(g, tr):
        bt = bt0 + g
        b, t = bt // _T, bt % _T
        rows = _ROWS_TR[tr]
        copies = []
        for s2 in range(_S):
            copies.append(
                pltpu.async_copy(
                    out_bufs[s2].at[pl.ds(0, rows)],
                    out_hbm.at[b, t, s2, pl.ds(tr * 8, rows)],
                    osem,
                )
            )
        return copies

    def gather_chunk(in_v, o0, no):
        @plsc.parallel_loop(0, _NJ, unroll=4)
        def j_body(j):
            tin = tin_v[pl.ds(j * _LANES, _LANES)]
            c0 = j * _LANES
            for u in range(no):
                r8 = (o0 + u) % 8
                for s2 in range(_S):
                    out_bufs[s2][r8, pl.ds(c0, _LANES)] = (
                        plsc.load_gather(
                            in_v, [tin + (u * _INW + s2 * 128)]
                        )
                    )

    out_copies = None
    in_copies = [None] * (4 * _BT_PER_W)
    in_copies[0] = start_in(0)
    for g in range(_BT_PER_W):
        for tr in range(2):
            for h in range(2):
                ci = g * 4 + tr * 2 + h
                if ci + 1 < 4 * _BT_PER_W:
                    in_copies[ci + 1] = start_in(ci + 1)
                in_copies[ci].wait()
                if h == 0 and out_copies is not None:
                    for cp in out_copies:
                        cp.wait()
                o0, no = _CHUNKS[tr * 2 + h]
                gather_chunk(in_bufs[ci % 2], o0, no)
            out_copies = start_out(g, tr)
    for cp in out_copies:
        cp.wait()


def kernel(inputs, sc_ind):
    x = inputs.transpose(0, 1, 3, 2, 4).reshape(_B, _T, _O, _S, _F // 128,
                                                128)
    x = x.transpose(0, 1, 2, 4, 3, 5).reshape(_NBT * _O * _INW)
    idx32 = sc_ind.astype(jnp.int32)
    idx = jnp.concatenate(
        [idx32, jnp.full((_NSCP - _NSC,), idx32[-1], jnp.int32)]
    )
    tin = (idx >> 7) * 256 + (idx & 127)
    mesh = plsc.VectorSubcoreMesh(core_axis_name="c", subcore_axis_name="s")
    out = pl.kernel(
        _gather_body,
        mesh=mesh,
        compiler_params=pltpu.CompilerParams(
            needs_layout_passes=False, use_tc_tiling_on_sc=True
        ),
        out_type=jax.ShapeDtypeStruct((_B, _T, _S, _O, _NSCP), jnp.float32),
        scratch_types=[
            pltpu.VMEM((_NJ * _LANES,), jnp.int32),
            pltpu.VMEM((4 * _INW,), jnp.float32),
            pltpu.VMEM((4 * _INW,), jnp.float32),
            pltpu.VMEM((8, _NSCP), jnp.float32),
            pltpu.VMEM((8, _NSCP), jnp.float32),
            pltpu.SemaphoreType.DMA,
            pltpu.SemaphoreType.DMA,
            pltpu.SemaphoreType.DMA,
        ],
    )(x, tin)
    return out[..., :_NSC]
